# baseline (speedup 1.0000x reference)
import functools

import jax
import jax.numpy as jnp
from jax import lax
from jax.experimental import pallas as pl
from jax.experimental.pallas import tpu as pltpu
from jax.experimental.pallas import tpu_sc as plsc

N = 4096
D = 512
KK = 32
BR = 256

NPAIR = N * KK
CHUNK_ROWS = 256
CHUNK_WORDS = CHUNK_ROWS * N
NCHUNK = N // CHUNK_ROWS
NSUB = 16
ZROWS = CHUNK_WORDS // NSUB
P2_PER_TILE = NPAIR // NSUB
P1_PER_TILE = CHUNK_ROWS * KK // NSUB
WIN = 8


def _topk_body(xb_ref, xa_ref, dist_ref, idx_ref):
    xb = xb_ref[...]
    xa = xa_ref[...]
    dot = jax.lax.dot_general(
        xb, xa, (((1,), (1,)), ((), ())),
        preferred_element_type=jnp.float32)
    sqb = jnp.sum(xb * xb, axis=1, keepdims=True)
    sqa = jnp.sum(xa * xa, axis=1, keepdims=True)
    d2 = jnp.maximum(sqb + sqa.T - 2.0 * dot, 0.0)
    iota = jax.lax.broadcasted_iota(jnp.int32, (BR, N), 1)
    vals = d2
    inf = jnp.float32(jnp.inf)
    dcols = []
    icols = []
    for _ in range(KK):
        m = jnp.min(vals, axis=1, keepdims=True)
        cand = jnp.where(vals == m, iota, N)
        ai = jnp.min(cand, axis=1, keepdims=True)
        vals = jnp.where(cand == ai, inf, vals)
        dcols.append(m)
        icols.append(ai)
    dist_ref[...] = jnp.sqrt(jnp.concatenate(dcols, axis=1))
    idx_ref[...] = jnp.concatenate(icols, axis=1)


def _topk(x):
    grid = N // BR
    return pl.pallas_call(
        _topk_body,
        grid=(grid,),
        in_specs=[
            pl.BlockSpec((BR, D), lambda i: (i, 0)),
            pl.BlockSpec((N, D), lambda i: (0, 0)),
        ],
        out_specs=[
            pl.BlockSpec((BR, KK), lambda i: (i, 0)),
            pl.BlockSpec((BR, KK), lambda i: (i, 0)),
        ],
        out_shape=[
            jax.ShapeDtypeStruct((N, KK), jnp.float32),
            jax.ShapeDtypeStruct((N, KK), jnp.int32),
        ],
    )(x, x)


def _adj_sc_body(cols_hbm, zeros_hbm, halfs_hbm, adj_hbm,
                 cols_t, cols_c, half, shared, dma_sem):
    core = lax.axis_index("c")
    sub = lax.axis_index("s")
    lanes = lax.iota(jnp.int32, 16)
    trash = jnp.int32(CHUNK_WORDS) + lanes

    pltpu.sync_copy(halfs_hbm, half)
    pltpu.sync_copy(cols_hbm.at[pl.ds(sub * P2_PER_TILE, P2_PER_TILE)],
                    cols_t)

    def chunk_body(ch, _):
        row0 = (core * (NCHUNK // 2) + ch) * CHUNK_ROWS
        row1 = row0 + CHUNK_ROWS
        my_off = sub * ZROWS

        pltpu.sync_copy(zeros_hbm, shared.at[pl.ds(my_off, ZROWS)])
        p1_base = row0 * KK + sub * P1_PER_TILE
        pltpu.sync_copy(cols_hbm.at[pl.ds(p1_base, P1_PER_TILE)], cols_c)

        plsc.subcore_barrier()

        def p1_body(it, _):
            p = p1_base + it * 16 + lanes
            r = lax.shift_right_logical(p, 5)
            c = cols_c[pl.ds(it * 16, 16)]
            addr = jnp.where(c >= 0, (r - row0) * N + c, trash)
            d = pltpu.async_copy(half, shared.at[addr], dma_sem, add=True)

            @pl.when(it >= WIN)
            def _():
                d.wait()
            return 0
        lax.fori_loop(0, P1_PER_TILE // 16, p1_body, 0)

        def p2_body(it, _):
            p = sub * P2_PER_TILE + it * 16 + lanes
            r = lax.shift_right_logical(p, 5)
            c = cols_t[pl.ds(it * 16, 16)]
            ok = (c >= row0) & (c < row1)
            addr = jnp.where(ok, (c - row0) * N + r, trash)
            d = pltpu.async_copy(half, shared.at[addr], dma_sem, add=True)
            d.wait()
            return 0
        n2 = P2_PER_TILE // 16
        lax.fori_loop(0, n2 - 1, p2_body, 0)
        it = n2 - 1
        p = sub * P2_PER_TILE + it * 16 + lanes
        r = lax.shift_right_logical(p, 5)
        c = cols_t[pl.ds(it * 16, 16)]
        ok = (c >= row0) & (c < row1)
        addr = jnp.where(ok, (c - row0) * N + r, trash)
        dd = pltpu.async_copy(half, shared.at[addr], dma_sem, add=True)
        for _ in range(WIN + 1):
            dd.wait()

        plsc.subcore_barrier()

        pltpu.sync_copy(
            shared.at[pl.ds(my_off, ZROWS)],
            adj_hbm.at[pl.ds(row0 * N + my_off, ZROWS)])
        return 0

    lax.fori_loop(0, NCHUNK // 2, chunk_body, 0)


def _adjacency_sc(cols_flat):
    mesh = plsc.VectorSubcoreMesh(core_axis_name="c", subcore_axis_name="s")
    f = functools.partial(
        pl.kernel,
        mesh=mesh,
        out_type=jax.ShapeDtypeStruct((N * N,), jnp.float32),
        scratch_types=[
            pltpu.VMEM((P2_PER_TILE,), jnp.int32),
            pltpu.VMEM((P1_PER_TILE,), jnp.int32),
            pltpu.VMEM((16,), jnp.float32),
            pltpu.VMEM_SHARED((CHUNK_WORDS + 16,), jnp.float32),
            pltpu.SemaphoreType.DMA,
        ],
    )(_adj_sc_body)
    zeros = jnp.zeros((ZROWS,), jnp.float32)
    halfs = jnp.full((16,), 0.5, jnp.float32)
    return f(cols_flat, zeros, halfs).reshape(N, N)


def kernel(x, k):
    dist, topi = _topk(x)
    valid = jnp.arange(KK, dtype=jnp.int32) < k
    distances = jnp.where(valid[None, :], dist, 0.0)
    topi_adj = jnp.where(valid[None, :], topi, -1).reshape(-1)
    adj = _adjacency_sc(topi_adj)
    return adj, distances, topi


# speedup vs baseline: 1.1139x; 1.1139x over previous
import functools

import jax
import jax.numpy as jnp
from jax import lax
from jax.experimental import pallas as pl
from jax.experimental.pallas import tpu as pltpu
from jax.experimental.pallas import tpu_sc as plsc

N = 4096
D = 512
KK = 32
BR = 512

NPAIR = N * KK
CHUNK_ROWS = 256
CHUNK_WORDS = CHUNK_ROWS * N
NCHUNK = N // CHUNK_ROWS
NSUB = 16
ZROWS = CHUNK_WORDS // NSUB
P2_PER_TILE = NPAIR // NSUB
P1_PER_TILE = CHUNK_ROWS * KK // NSUB
WIN = 8


def _topk_body(xb_ref, xa_ref, dist_ref, idx_ref):
    xb = xb_ref[...]
    xa = xa_ref[...]
    dot = jax.lax.dot_general(
        xb, xa, (((1,), (1,)), ((), ())),
        preferred_element_type=jnp.float32)
    sqb = jnp.sum(xb * xb, axis=1, keepdims=True)
    sqa = jnp.sum(xa * xa, axis=1, keepdims=True)
    d2 = jnp.maximum(sqb + sqa.T - 2.0 * dot, 0.0)
    iota = jax.lax.broadcasted_iota(jnp.int32, (BR, N), 1)
    vals = d2
    inf = jnp.float32(jnp.inf)
    dcols = []
    icols = []
    for _ in range(KK):
        m = jnp.min(vals, axis=1, keepdims=True)
        cand = jnp.where(vals == m, iota, N)
        ai = jnp.min(cand, axis=1, keepdims=True)
        vals = jnp.where(cand == ai, inf, vals)
        dcols.append(m)
        icols.append(ai)
    dist_ref[...] = jnp.sqrt(jnp.concatenate(dcols, axis=1))
    idx_ref[...] = jnp.concatenate(icols, axis=1)


def _topk(x):
    grid = N // BR
    return pl.pallas_call(
        _topk_body,
        grid=(grid,),
        in_specs=[
            pl.BlockSpec((BR, D), lambda i: (i, 0)),
            pl.BlockSpec((N, D), lambda i: (0, 0)),
        ],
        out_specs=[
            pl.BlockSpec((BR, KK), lambda i: (i, 0)),
            pl.BlockSpec((BR, KK), lambda i: (i, 0)),
        ],
        out_shape=[
            jax.ShapeDtypeStruct((N, KK), jnp.float32),
            jax.ShapeDtypeStruct((N, KK), jnp.int32),
        ],
    )(x, x)


def _adj_sc_body(cols_hbm, zeros_hbm, halfs_hbm, adj_hbm,
                 cols_t, cols_c, half, shared, dma_sem):
    core = lax.axis_index("c")
    sub = lax.axis_index("s")
    lanes = lax.iota(jnp.int32, 16)
    trash = jnp.int32(CHUNK_WORDS) + lanes

    pltpu.sync_copy(halfs_hbm, half)
    pltpu.sync_copy(cols_hbm.at[pl.ds(sub * P2_PER_TILE, P2_PER_TILE)],
                    cols_t)

    def chunk_body(ch, _):
        row0 = (core * (NCHUNK // 2) + ch) * CHUNK_ROWS
        row1 = row0 + CHUNK_ROWS
        my_off = sub * ZROWS

        pltpu.sync_copy(zeros_hbm, shared.at[pl.ds(my_off, ZROWS)])
        p1_base = row0 * KK + sub * P1_PER_TILE
        pltpu.sync_copy(cols_hbm.at[pl.ds(p1_base, P1_PER_TILE)], cols_c)

        plsc.subcore_barrier()

        def p1_body(it, _):
            p = p1_base + it * 16 + lanes
            r = lax.shift_right_logical(p, 5)
            c = cols_c[pl.ds(it * 16, 16)]
            addr = jnp.where(c >= 0, (r - row0) * N + c, trash)
            d = pltpu.async_copy(half, shared.at[addr], dma_sem, add=True)

            @pl.when(it >= WIN)
            def _():
                d.wait()
            return 0
        lax.fori_loop(0, P1_PER_TILE // 16, p1_body, 0)

        def p2_body(it, _):
            p = sub * P2_PER_TILE + it * 16 + lanes
            r = lax.shift_right_logical(p, 5)
            c = cols_t[pl.ds(it * 16, 16)]
            ok = (c >= row0) & (c < row1)
            addr = jnp.where(ok, (c - row0) * N + r, trash)
            d = pltpu.async_copy(half, shared.at[addr], dma_sem, add=True)
            d.wait()
            return 0
        n2 = P2_PER_TILE // 16
        lax.fori_loop(0, n2 - 1, p2_body, 0)
        it = n2 - 1
        p = sub * P2_PER_TILE + it * 16 + lanes
        r = lax.shift_right_logical(p, 5)
        c = cols_t[pl.ds(it * 16, 16)]
        ok = (c >= row0) & (c < row1)
        addr = jnp.where(ok, (c - row0) * N + r, trash)
        dd = pltpu.async_copy(half, shared.at[addr], dma_sem, add=True)
        for _ in range(WIN + 1):
            dd.wait()

        plsc.subcore_barrier()

        pltpu.sync_copy(
            shared.at[pl.ds(my_off, ZROWS)],
            adj_hbm.at[pl.ds(row0 * N + my_off, ZROWS)])
        return 0

    lax.fori_loop(0, NCHUNK // 2, chunk_body, 0)


def _adjacency_sc(cols_flat):
    mesh = plsc.VectorSubcoreMesh(core_axis_name="c", subcore_axis_name="s")
    f = functools.partial(
        pl.kernel,
        mesh=mesh,
        out_type=jax.ShapeDtypeStruct((N * N,), jnp.float32),
        scratch_types=[
            pltpu.VMEM((P2_PER_TILE,), jnp.int32),
            pltpu.VMEM((P1_PER_TILE,), jnp.int32),
            pltpu.VMEM((16,), jnp.float32),
            pltpu.VMEM_SHARED((CHUNK_WORDS + 16,), jnp.float32),
            pltpu.SemaphoreType.DMA,
        ],
    )(_adj_sc_body)
    zeros = jnp.zeros((ZROWS,), jnp.float32)
    halfs = jnp.full((16,), 0.5, jnp.float32)
    return f(cols_flat, zeros, halfs).reshape(N, N)


def kernel(x, k):
    dist, topi = _topk(x)
    valid = jnp.arange(KK, dtype=jnp.int32) < k
    distances = jnp.where(valid[None, :], dist, 0.0)
    topi_adj = jnp.where(valid[None, :], topi, -1).reshape(-1)
    adj = _adjacency_sc(topi_adj)
    return adj, distances, topi


# speedup vs baseline: 1.1148x; 1.0008x over previous
import functools

import jax
import jax.numpy as jnp
from jax import lax
from jax.experimental import pallas as pl
from jax.experimental.pallas import tpu as pltpu
from jax.experimental.pallas import tpu_sc as plsc

N = 4096
D = 512
KK = 32
BR = 512

NPAIR = N * KK
CHUNK_ROWS = 256
CHUNK_WORDS = CHUNK_ROWS * N
NCHUNK = N // CHUNK_ROWS
NSUB = 16
ZROWS = CHUNK_WORDS // NSUB
P2_PER_TILE = NPAIR // NSUB
P1_PER_TILE = CHUNK_ROWS * KK // NSUB
WIN = 16


def _topk_body(xb_ref, xa_ref, dist_ref, idx_ref):
    xb = xb_ref[...]
    xa = xa_ref[...]
    dot = jax.lax.dot_general(
        xb, xa, (((1,), (1,)), ((), ())),
        preferred_element_type=jnp.float32)
    sqb = jnp.sum(xb * xb, axis=1, keepdims=True)
    sqa = jnp.sum(xa * xa, axis=1, keepdims=True)
    d2 = jnp.maximum(sqb + sqa.T - 2.0 * dot, 0.0)
    iota = jax.lax.broadcasted_iota(jnp.int32, (BR, N), 1)
    vals = d2
    inf = jnp.float32(jnp.inf)
    dcols = []
    icols = []
    for _ in range(KK):
        m = jnp.min(vals, axis=1, keepdims=True)
        cand = jnp.where(vals == m, iota, N)
        ai = jnp.min(cand, axis=1, keepdims=True)
        vals = jnp.where(cand == ai, inf, vals)
        dcols.append(m)
        icols.append(ai)
    dist_ref[...] = jnp.sqrt(jnp.concatenate(dcols, axis=1))
    idx_ref[...] = jnp.concatenate(icols, axis=1)


def _topk(x):
    grid = N // BR
    return pl.pallas_call(
        _topk_body,
        grid=(grid,),
        in_specs=[
            pl.BlockSpec((BR, D), lambda i: (i, 0)),
            pl.BlockSpec((N, D), lambda i: (0, 0)),
        ],
        out_specs=[
            pl.BlockSpec((BR, KK), lambda i: (i, 0)),
            pl.BlockSpec((BR, KK), lambda i: (i, 0)),
        ],
        out_shape=[
            jax.ShapeDtypeStruct((N, KK), jnp.float32),
            jax.ShapeDtypeStruct((N, KK), jnp.int32),
        ],
    )(x, x)


def _adj_sc_body(cols_hbm, zeros_hbm, halfs_hbm, adj_hbm,
                 cols_t, cols_c, half, shared, dma_sem):
    core = lax.axis_index("c")
    sub = lax.axis_index("s")
    lanes = lax.iota(jnp.int32, 16)
    trash = jnp.int32(CHUNK_WORDS) + lanes

    pltpu.sync_copy(halfs_hbm, half)
    pltpu.sync_copy(cols_hbm.at[pl.ds(sub * P2_PER_TILE, P2_PER_TILE)],
                    cols_t)

    def chunk_body(ch, _):
        row0 = (core * (NCHUNK // 2) + ch) * CHUNK_ROWS
        row1 = row0 + CHUNK_ROWS
        my_off = sub * ZROWS

        pltpu.sync_copy(zeros_hbm, shared.at[pl.ds(my_off, ZROWS)])
        p1_base = row0 * KK + sub * P1_PER_TILE
        pltpu.sync_copy(cols_hbm.at[pl.ds(p1_base, P1_PER_TILE)], cols_c)

        plsc.subcore_barrier()

        def p1_body(it, _):
            p = p1_base + it * 16 + lanes
            r = lax.shift_right_logical(p, 5)
            c = cols_c[pl.ds(it * 16, 16)]
            addr = jnp.where(c >= 0, (r - row0) * N + c, trash)
            d = pltpu.async_copy(half, shared.at[addr], dma_sem, add=True)

            @pl.when(it >= WIN)
            def _():
                d.wait()
            return 0
        lax.fori_loop(0, P1_PER_TILE // 16, p1_body, 0)

        def p2_body(it, _):
            p = sub * P2_PER_TILE + it * 16 + lanes
            r = lax.shift_right_logical(p, 5)
            c = cols_t[pl.ds(it * 16, 16)]
            ok = (c >= row0) & (c < row1)
            addr = jnp.where(ok, (c - row0) * N + r, trash)

            d = pltpu.async_copy(half, shared.at[addr], dma_sem, add=True)
            d.wait()
            return 0
        n2 = P2_PER_TILE // 16
        lax.fori_loop(0, n2 - 1, p2_body, 0)
        it = n2 - 1
        p = sub * P2_PER_TILE + it * 16 + lanes
        r = lax.shift_right_logical(p, 5)
        c = cols_t[pl.ds(it * 16, 16)]
        ok = (c >= row0) & (c < row1)
        addr = jnp.where(ok, (c - row0) * N + r, trash)
        dd = pltpu.async_copy(half, shared.at[addr], dma_sem, add=True)
        for _ in range(WIN + 1):
            dd.wait()

        plsc.subcore_barrier()

        pltpu.sync_copy(
            shared.at[pl.ds(my_off, ZROWS)],
            adj_hbm.at[pl.ds(row0 * N + my_off, ZROWS)])
        return 0

    lax.fori_loop(0, NCHUNK // 2, chunk_body, 0)


def _adjacency_sc(cols_flat):
    mesh = plsc.VectorSubcoreMesh(core_axis_name="c", subcore_axis_name="s")
    f = functools.partial(
        pl.kernel,
        mesh=mesh,
        out_type=jax.ShapeDtypeStruct((N * N,), jnp.float32),
        scratch_types=[
            pltpu.VMEM((P2_PER_TILE,), jnp.int32),
            pltpu.VMEM((P1_PER_TILE,), jnp.int32),
            pltpu.VMEM((16,), jnp.float32),
            pltpu.VMEM_SHARED((CHUNK_WORDS + 16,), jnp.float32),
            pltpu.SemaphoreType.DMA,
        ],
    )(_adj_sc_body)
    zeros = jnp.zeros((ZROWS,), jnp.float32)
    halfs = jnp.full((16,), 0.5, jnp.float32)
    return f(cols_flat, zeros, halfs).reshape(N, N)


def kernel(x, k):
    dist, topi = _topk(x)
    valid = jnp.arange(KK, dtype=jnp.int32) < k
    distances = jnp.where(valid[None, :], dist, 0.0)
    topi_adj = jnp.where(valid[None, :], topi, -1).reshape(-1)
    adj = _adjacency_sc(topi_adj)
    return adj, distances, topi
